# Initial kernel scaffold; baseline (speedup 1.0000x reference)
import jax
import jax.numpy as jnp
from jax.experimental import pallas as pl


def kernel(x_s, x_t, edge_index, edge_attr, u, batch_e, batch_s, batch_t, params):
    raise NotImplementedError("write your pallas kernel here")


# speedup vs baseline: 2.5209x; 2.5209x over previous
import functools

import jax
import jax.numpy as jnp
from jax import lax
from jax.experimental import pallas as pl
from jax.experimental.pallas import tpu as pltpu
from jax.experimental.pallas import tpu_sc as plsc

NW = 32
RJ = 80
KJ = 5
CH = KJ * RJ
BE = 6400
BN = 4000


def _leaky(x):
    return jnp.where(x >= 0, x, 0.1 * x)


def _onehot16(idx_col):
    return (idx_col == lax.broadcasted_iota(jnp.int32, (1, 16), 1)).astype(jnp.float32)


def _sc_gather2(xs_tbl, xt_tbl, src, tgt):
    E = src.shape[0]
    DS = xs_tbl.shape[1]
    DT = xt_tbl.shape[1]
    perw = E // NW
    nit = perw // CH
    mesh = plsc.VectorSubcoreMesh(core_axis_name="c", subcore_axis_name="s")

    @functools.partial(
        pl.kernel,
        out_type=(jax.ShapeDtypeStruct((E, DS), jnp.float32),
                  jax.ShapeDtypeStruct((E, DT), jnp.float32)),
        mesh=mesh,
        compiler_params=pltpu.CompilerParams(use_tc_tiling_on_sc=False),
        scratch_types=[pltpu.VMEM((CH,), jnp.int32),
                       pltpu.VMEM((CH,), jnp.int32),
                       pltpu.VMEM((CH, DS), jnp.float32),
                       pltpu.VMEM((CH, DT), jnp.float32),
                       pltpu.SemaphoreType.DMA,
                       pltpu.SemaphoreType.DMA],
    )
    def k(xs_hbm, xt_hbm, src_hbm, tgt_hbm, oxs_hbm, oxt_hbm,
          si_v, ti_v, rs_v, rt_v, sem_s, sem_t):
        wid = lax.axis_index("s") * 2 + lax.axis_index("c")
        base = wid * perw

        def body(i, _):
            off = base + i * CH
            pltpu.sync_copy(src_hbm.at[pl.ds(off, CH)], si_v)
            pltpu.sync_copy(tgt_hbm.at[pl.ds(off, CH)], ti_v)
            ds = []
            for j in range(KJ):
                sl = pl.ds(j * RJ, RJ)
                ds.append(pltpu.async_copy(
                    xs_hbm.at[si_v.at[sl]], rs_v.at[sl], sem_s))
                ds.append(pltpu.async_copy(
                    xt_hbm.at[ti_v.at[sl]], rt_v.at[sl], sem_t))
            for d in ds:
                d.wait()
            pltpu.sync_copy(rs_v, oxs_hbm.at[pl.ds(off, CH)])
            pltpu.sync_copy(rt_v, oxt_hbm.at[pl.ds(off, CH)])
            return 0

        lax.fori_loop(0, nit, body, 0)

    return k(xs_tbl, xt_tbl, src, tgt)


def _sc_gather1(tbl, idx):
    E = idx.shape[0]
    D = tbl.shape[1]
    perw = E // NW
    nit = perw // CH
    mesh = plsc.VectorSubcoreMesh(core_axis_name="c", subcore_axis_name="s")

    @functools.partial(
        pl.kernel,
        out_type=jax.ShapeDtypeStruct((E, D), jnp.float32),
        mesh=mesh,
        compiler_params=pltpu.CompilerParams(use_tc_tiling_on_sc=False),
        scratch_types=[pltpu.VMEM((CH,), jnp.int32),
                       pltpu.VMEM((CH, D), jnp.float32),
                       pltpu.SemaphoreType.DMA],
    )
    def k(tbl_hbm, idx_hbm, out_hbm, i_v, r_v, sem):
        wid = lax.axis_index("s") * 2 + lax.axis_index("c")
        base = wid * perw

        def body(i, _):
            off = base + i * CH
            pltpu.sync_copy(idx_hbm.at[pl.ds(off, CH)], i_v)
            ds = []
            for j in range(KJ):
                sl = pl.ds(j * RJ, RJ)
                ds.append(pltpu.async_copy(tbl_hbm.at[i_v.at[sl]],
                                           r_v.at[sl], sem))
            for d in ds:
                d.wait()
            pltpu.sync_copy(r_v, out_hbm.at[pl.ds(off, CH)])
            return 0

        lax.fori_loop(0, nit, body, 0)

    return k(tbl, idx)


SKJ = 8
SCH = SKJ * RJ
SNIT = 78
SPAD = 632


def _sc_scatter_sum(rows_list, idx3, nseg, width):
    M = len(rows_list)
    E = rows_list[0].shape[0]
    perw = E // NW
    zslc = 6256 if nseg == 100000 else nseg // 16
    mesh = plsc.VectorSubcoreMesh(core_axis_name="c", subcore_axis_name="s")
    zer = jnp.zeros((nseg, width), jnp.float32)

    @functools.partial(
        pl.kernel,
        out_type=jax.ShapeDtypeStruct((M, 2, nseg, width), jnp.float32),
        mesh=mesh,
        compiler_params=pltpu.CompilerParams(use_tc_tiling_on_sc=False),
        scratch_types=[pltpu.VMEM((SKJ, RJ), jnp.int32),
                       pltpu.VMEM((SCH, width), jnp.float32),
                       pltpu.VMEM_SHARED((nseg, width), jnp.float32),
                       pltpu.SemaphoreType.DMA],
    )
    def k(*refs):
        rows_hbms = refs[:M]
        zer_hbm = refs[M]
        idx_hbm = refs[M + 1]
        out_hbm = refs[M + 2]
        i_v, r_v, acc, sem = refs[M + 3:]
        cid = lax.axis_index("c")
        sid = lax.axis_index("s")
        wid = sid * 2 + cid
        base = wid * perw
        zoff = jnp.minimum(sid * zslc, nseg - zslc)
        for m in range(M):
            pltpu.sync_copy(zer_hbm.at[pl.ds(zoff, zslc)],
                            acc.at[pl.ds(zoff, zslc)])
            plsc.subcore_barrier()

            def body(i, _):
                pltpu.sync_copy(idx_hbm.at[wid, pl.ds(i * SKJ, SKJ)], i_v)
                pltpu.sync_copy(rows_hbms[m].at[pl.ds(base + i * SCH, SCH)],
                                r_v)
                for j in range(SKJ):
                    pltpu.sync_copy(r_v.at[pl.ds(j * RJ, RJ)],
                                    acc.at[i_v.at[j]], add=True)
                return 0

            lax.fori_loop(0, SNIT, body, 0)
            pltpu.sync_copy(idx_hbm.at[wid, pl.ds(SNIT * SKJ, SKJ)], i_v)
            pltpu.sync_copy(
                rows_hbms[m].at[pl.ds(base + SNIT * SCH, RJ)],
                r_v.at[pl.ds(0, RJ)])
            pltpu.sync_copy(r_v.at[pl.ds(0, RJ)], acc.at[i_v.at[0]],
                            add=True)
            plsc.subcore_barrier()
            pltpu.sync_copy(acc.at[pl.ds(zoff, zslc)],
                            out_hbm.at[m, cid, pl.ds(zoff, zslc)])
            plsc.subcore_barrier()

    return k(*rows_list, zer, idx3)


def _tc_edge(gxs, gxt, ea, be1, u, pe, ps1, aff):
    E = gxs.shape[0]
    G = E // BE

    def body(gxs_r, gxt_r, ea_r, be_r, u_r, w1_r, c1_r, w2_r, c2_r,
             v1_r, d1_r, v2_r, d2_r, aff_r,
             eap_r, p1_r, p2_r, bp_r):
        ea_b = ea_r[...] * aff_r[0:1, :] + aff_r[1:2, :]
        ue = _onehot16(be_r[...]) @ u_r[...]
        x = gxs_r[:, 0:10]
        xt = gxt_r[:, 0:5]
        h = (x @ w1_r[0:10, :] + xt @ w1_r[10:15, :] + ea_b @ w1_r[15:25, :]
             + ue @ w1_r[25:35, :] + c1_r[...])
        eap = _leaky(h) @ w2_r[...] + c2_r[...]
        eap_r[...] = eap
        bp_r[0, 0, :] = jnp.sum(eap, 0)
        bp_r[0, 1, :] = jnp.sum(eap * eap, 0)
        sh = _leaky(xt @ v1_r[0:5, :] + eap @ v1_r[5:15, :] + d1_r[...])
        s_out = sh @ v2_r[...] + d2_r[...]
        p1 = jnp.concatenate([s_out, jnp.ones((BE, 1), jnp.float32)], axis=1)
        p1_r[...] = p1
        p2_r[...] = p1 * p1

    full = lambda s: pl.BlockSpec(s, lambda i: (0,) * len(s))
    row = lambda w: pl.BlockSpec((BE, w), lambda i: (i, 0))
    out = pl.pallas_call(
        body,
        grid=(G,),
        in_specs=[row(32), row(32), row(10), row(1), full((16, 10)),
                  full((35, 10)), full((1, 10)), full((10, 10)), full((1, 10)),
                  full((15, 15)), full((1, 15)), full((15, 15)), full((1, 15)),
                  full((2, 10))],
        out_specs=[row(10), row(16), row(16),
                   pl.BlockSpec((1, 2, 10), lambda i: (i, 0, 0))],
        out_shape=[jax.ShapeDtypeStruct((E, 10), jnp.float32)]
                  + [jax.ShapeDtypeStruct((E, 16), jnp.float32)] * 2
                  + [jax.ShapeDtypeStruct((G, 2, 10), jnp.float32)],
    )(gxs, gxt, ea, be1, u,
      pe['l1']['W'], pe['l1']['b'].reshape(1, 10), pe['l2']['W'],
      pe['l2']['b'].reshape(1, 10),
      ps1['l1']['W'], ps1['l1']['b'].reshape(1, 15), ps1['l2']['W'],
      ps1['l2']['b'].reshape(1, 15), aff)
    return out


def _tc_amean(mom1):
    NS = mom1.shape[1]
    G = NS // BN

    def body(m_r, a_r):
        m = m_r[0] + m_r[1]
        n = m[:, 15:16]
        a_r[...] = jnp.concatenate(
            [m[:, 0:15] / jnp.maximum(n, 1.0), n,
             jnp.zeros((BN, 16), jnp.float32)], axis=1)

    return pl.pallas_call(
        body,
        grid=(G,),
        in_specs=[pl.BlockSpec((2, BN, 16), lambda i: (0, i, 0))],
        out_specs=pl.BlockSpec((BN, 32), lambda i: (i, 0)),
        out_shape=jax.ShapeDtypeStruct((NS, 32), jnp.float32),
    )(mom1)


def _tc_center(p1, arows):
    E = p1.shape[0]
    G = E // BE

    def body(p_r, a_r, q3_r, q4_r):
        cen = p_r[...] - a_r[:, 0:16]
        c2 = cen * cen
        q3_r[...] = c2 * cen
        q4_r[...] = c2 * c2

    row = lambda w: pl.BlockSpec((BE, w), lambda i: (i, 0))
    return pl.pallas_call(
        body,
        grid=(G,),
        in_specs=[row(16), row(32)],
        out_specs=[row(16), row(16)],
        out_shape=[jax.ShapeDtypeStruct((E, 16), jnp.float32)] * 2,
    )(p1, arows)


def _tc_smodel(xs, a_tbl, m2p, q3p, q4p, bs1, u, ps2):
    NS = xs.shape[0]
    G = NS // BN

    def body(xs_r, a_r, m2_r, q3_r, q4_r, bs_r, u_r, w1_r, c1_r, w2_r, c2_r,
             pre_r, st_r, gp_r):
        i = pl.program_id(0)

        @pl.when(i == 0)
        def _():
            st_r[...] = jnp.zeros((2, 16), jnp.float32)
            gp_r[...] = jnp.zeros((16, 16), jnp.float32)

        n = a_r[:, 15:16]
        nc = jnp.maximum(n, 1.0)
        a = a_r[:, 0:15]
        x = xs_r[:, 0:10]
        m2 = (m2_r[0] + m2_r[1])[:, 0:15] / nc
        m3 = (q3_r[0] + q3_r[1])[:, 0:15] / nc
        m4 = (q4_r[0] + q4_r[1])[:, 0:15] / nc
        b = jnp.sqrt(1e-6 + jnp.maximum(m2 - a * a, 0.0))
        c = m3 / (b * b * b)
        d = m4 / ((b * b) * (b * b))
        oh = _onehot16(bs_r[...])
        ue = oh @ u_r[...]
        h = (x @ w1_r[0:10, :] + n @ w1_r[10:11, :] + a @ w1_r[11:26, :]
             + b @ w1_r[26:41, :] + c @ w1_r[41:56, :] + d @ w1_r[56:71, :]
             + ue @ w1_r[71:81, :] + c1_r[...])
        xp = _leaky(h) @ w2_r[...] + c2_r[...]
        pre_r[...] = jnp.concatenate(
            [xp, jnp.zeros((BN, 22), jnp.float32)], axis=1)
        st_r[0:1, 0:10] = st_r[0:1, 0:10] + jnp.sum(xp, 0, keepdims=True)
        st_r[1:2, 0:10] = st_r[1:2, 0:10] + jnp.sum(xp * xp, 0, keepdims=True)
        gp_r[:, 0:10] = gp_r[:, 0:10] + lax.dot_general(
            oh, xp, (((0,), (0,)), ((), ())),
            preferred_element_type=jnp.float32)
        gp_r[:, 10:11] = gp_r[:, 10:11] + jnp.sum(oh, 0)[:, None]

    full = lambda s: pl.BlockSpec(s, lambda i: (0,) * len(s))
    row = lambda w: pl.BlockSpec((BN, w), lambda i: (i, 0))
    mom_s = pl.BlockSpec((2, BN, 16), lambda i: (0, i, 0))
    return pl.pallas_call(
        body,
        grid=(G,),
        in_specs=[row(32), row(32), mom_s, mom_s, mom_s, row(1),
                  full((16, 10)),
                  full((81, 10)), full((1, 10)), full((10, 10)), full((1, 10))],
        out_specs=[row(32), full((2, 16)), full((16, 16))],
        out_shape=[jax.ShapeDtypeStruct((NS, 32), jnp.float32),
                   jax.ShapeDtypeStruct((2, 16), jnp.float32),
                   jax.ShapeDtypeStruct((16, 16), jnp.float32)],
    )(xs, a_tbl, m2p, q3p, q4p, bs1, u,
      ps2['l1']['W'], ps2['l1']['b'].reshape(1, 10), ps2['l2']['W'],
      ps2['l2']['b'].reshape(1, 10))


def _tc_bn_apply(xp, aff):
    N = xp.shape[0]
    W = aff.shape[1]
    G = N // BN

    def body(x_r, a_r, o_r):
        v = x_r[:, 0:W] * a_r[0:1, :] + a_r[1:2, :]
        o_r[...] = jnp.concatenate(
            [v, jnp.zeros((BN, 32 - W), jnp.float32)], axis=1)

    return pl.pallas_call(
        body,
        grid=(G,),
        in_specs=[pl.BlockSpec((BN, 32), lambda i: (i, 0)),
                  pl.BlockSpec((2, W), lambda i: (0, 0))],
        out_specs=pl.BlockSpec((BN, 32), lambda i: (i, 0)),
        out_shape=jax.ShapeDtypeStruct((N, 32), jnp.float32),
    )(xp, aff)


def _tc_t1(gxs2, eap, pt1):
    E = gxs2.shape[0]
    G = E // BE

    def body(x_r, e_r, w1_r, c1_r, w2_r, c2_r, o1_r, o2_r):
        h = _leaky(x_r[:, 0:10] @ w1_r[0:10, :] + e_r[...] @ w1_r[10:20, :]
                   + c1_r[...])
        o = h @ w2_r[...] + c2_r[...]
        o1_r[...] = o[:, 0:10]
        o2_r[...] = o[:, 10:20]

    full = lambda s: pl.BlockSpec(s, lambda i: (0,) * len(s))
    row = lambda w: pl.BlockSpec((BE, w), lambda i: (i, 0))
    return pl.pallas_call(
        body,
        grid=(G,),
        in_specs=[row(32), row(10), full((20, 20)), full((1, 20)),
                  full((20, 20)), full((1, 20))],
        out_specs=[row(10), row(10)],
        out_shape=[jax.ShapeDtypeStruct((E, 10), jnp.float32)] * 2,
    )(gxs2, eap, pt1['l1']['W'], pt1['l1']['b'].reshape(1, 20),
      pt1['l2']['W'], pt1['l2']['b'].reshape(1, 20))


def _tc_tmodel(xt, tsum1, tsum2, bt1, u, pt2):
    NT = xt.shape[0]
    G = NT // BN

    def body(xt_r, ts1_r, ts2_r, bt_r, u_r, w1_r, c1_r, w2_r, c2_r,
             pre_r, st_r, gp_r):
        i = pl.program_id(0)

        @pl.when(i == 0)
        def _():
            st_r[...] = jnp.zeros((2, 8), jnp.float32)
            gp_r[...] = jnp.zeros((16, 8), jnp.float32)

        a = jnp.concatenate([ts1_r[0] + ts1_r[1], ts2_r[0] + ts2_r[1]],
                            axis=1)
        oh = _onehot16(bt_r[...])
        ue = oh @ u_r[...]
        x = xt_r[:, 0:5]
        h = (x @ w1_r[0:5, :] + a @ w1_r[5:25, :] + ue @ w1_r[25:35, :]
             + c1_r[...])
        xp = _leaky(h) @ w2_r[...] + c2_r[...]
        pre_r[...] = jnp.concatenate(
            [xp, jnp.zeros((BN, 27), jnp.float32)], axis=1)
        st_r[0:1, 0:5] = st_r[0:1, 0:5] + jnp.sum(xp, 0, keepdims=True)
        st_r[1:2, 0:5] = st_r[1:2, 0:5] + jnp.sum(xp * xp, 0, keepdims=True)
        gp_r[:, 0:5] = gp_r[:, 0:5] + lax.dot_general(
            oh, xp, (((0,), (0,)), ((), ())),
            preferred_element_type=jnp.float32)
        gp_r[:, 5:6] = gp_r[:, 5:6] + jnp.sum(oh, 0)[:, None]

    full = lambda s: pl.BlockSpec(s, lambda i: (0,) * len(s))
    row = lambda w: pl.BlockSpec((BN, w), lambda i: (i, 0))
    return pl.pallas_call(
        body,
        grid=(G,),
        in_specs=[row(32), pl.BlockSpec((2, BN, 10), lambda i: (0, i, 0)),
                  pl.BlockSpec((2, BN, 10), lambda i: (0, i, 0)),
                  row(1), full((16, 10)),
                  full((35, 5)), full((1, 5)), full((5, 5)), full((1, 5))],
        out_specs=[row(32), full((2, 8)), full((16, 8))],
        out_shape=[jax.ShapeDtypeStruct((NT, 32), jnp.float32),
                   jax.ShapeDtypeStruct((2, 8), jnp.float32),
                   jax.ShapeDtypeStruct((16, 8), jnp.float32)],
    )(xt, tsum1, tsum2, bt1, u,
      pt2['l1']['W'], pt2['l1']['b'].reshape(1, 5), pt2['l2']['W'],
      pt2['l2']['b'].reshape(1, 5))


def _tc_gmodel(u, gpart_s, gpart_t, pg):
    def body(u_r, gs_r, gt_r, g1_r, e1_r, g2_r, e2_r, uo_r):
        ms = gs_r[:, 0:10] / jnp.maximum(gs_r[:, 10:11], 1.0)
        mt = gt_r[:, 0:5] / jnp.maximum(gt_r[:, 5:6], 1.0)
        hg = (u_r[...] @ g1_r[0:10, :] + ms @ g1_r[10:20, :]
              + mt @ g1_r[20:25, :] + e1_r[...])
        uo_r[...] = _leaky(hg) @ g2_r[...] + e2_r[...]

    return pl.pallas_call(
        body,
        out_shape=jax.ShapeDtypeStruct((16, 10), jnp.float32),
    )(u, gpart_s, gpart_t,
      pg['l1']['W'], pg['l1']['b'].reshape(1, 10), pg['l2']['W'],
      pg['l2']['b'].reshape(1, 10))


def _tc_final(gxs, gxt, ea, be1, u, pl_last, aff, noise):
    E = gxs.shape[0]
    G = E // BE

    def body(gxs_r, gxt_r, ea_r, be_r, u_r, w1_r, c1_r, w2_r, c2_r,
             aff_r, nz_r, t_r):
        ea_b = ea_r[...] * aff_r[0:1, :] + aff_r[1:2, :]
        ue = _onehot16(be_r[...]) @ u_r[...]
        h = (gxs_r[:, 0:10] @ w1_r[0:10, :] + gxt_r[:, 0:5] @ w1_r[10:15, :]
             + ea_b @ w1_r[15:25, :] + ue @ w1_r[25:35, :] + c1_r[...])
        o = _leaky(h) @ w2_r[...] + c2_r[...]
        mx = jnp.max(o, axis=1, keepdims=True)
        e = jnp.exp(o - mx)
        cls = lax.broadcasted_iota(jnp.int32, (1, 5), 1).astype(jnp.float32)
        t = jnp.sum(e * cls, axis=1, keepdims=True) \
            / jnp.sum(e, axis=1, keepdims=True)
        t = t + nz_r[...]
        it = jnp.floor(t)
        z = 20.0 * (t - 0.5 - it)
        t_r[...] = it + 1.0 / (1.0 + jnp.exp(-z))

    full = lambda s: pl.BlockSpec(s, lambda i: (0,) * len(s))
    row = lambda w: pl.BlockSpec((BE, w), lambda i: (i, 0))
    return pl.pallas_call(
        body,
        grid=(G,),
        in_specs=[row(32), row(32), row(10), row(1), full((16, 10)),
                  full((35, 5)), full((1, 5)), full((5, 5)), full((1, 5)),
                  full((2, 10)), row(1)],
        out_specs=row(1),
        out_shape=jax.ShapeDtypeStruct((E, 1), jnp.float32),
    )(gxs, gxt, ea, be1, u, pl_last['l1']['W'],
      pl_last['l1']['b'].reshape(1, 5), pl_last['l2']['W'],
      pl_last['l2']['b'].reshape(1, 5), aff, noise)


def _bn_affine(stats, g, b, n):
    w = g.shape[0]
    mean = stats[0, :w] / n
    var = stats[1, :w] / n - mean * mean
    sc = g * lax.rsqrt(var + 1e-5)
    return jnp.stack([sc, b - mean * sc])


def kernel(x_s, x_t, edge_index, edge_attr, u, batch_e, batch_s, batch_t,
           params):
    E = edge_attr.shape[0]
    NS = x_s.shape[0]
    NT = x_t.shape[0]
    src = edge_index[0]
    tgt = edge_index[1]
    nrow = E // NW // RJ
    sidx3 = jnp.pad(src.reshape(NW, nrow, RJ),
                    ((0, 0), (0, SPAD - nrow), (0, 0)))
    tidx3 = jnp.pad(tgt.reshape(NW, nrow, RJ),
                    ((0, 0), (0, SPAD - nrow), (0, 0)))
    be1 = batch_e.reshape(E, 1)
    bs1 = batch_s.reshape(NS, 1)
    bt1 = batch_t.reshape(NT, 1)
    noise = (0.3 * (jax.random.uniform(jax.random.key(1234), (E,),
                                       jnp.float32) - 0.5)).reshape(E, 1)

    ea = edge_attr
    aff = jnp.stack([jnp.ones((10,), jnp.float32),
                     jnp.zeros((10,), jnp.float32)])
    xs_cur = jnp.pad(x_s, ((0, 0), (0, 22)))
    xt_cur = jnp.pad(x_t, ((0, 0), (0, 27)))
    u_cur = u

    for i in range(4):
        p = params['blocks'][i]
        bn = params['bns'][i]
        gxs, gxt = _sc_gather2(xs_cur, xt_cur, src, tgt)
        eap, p1, p2, bpart = _tc_edge(
            gxs, gxt, ea, be1, u_cur, p['edge'], p['s1'], aff)
        mom12 = _sc_scatter_sum([p1, p2], sidx3, NS, 16)
        a_tbl = _tc_amean(mom12[0])
        arows = _sc_gather1(a_tbl, src)
        q3, q4 = _tc_center(p1, arows)
        mom34 = _sc_scatter_sum([q3, q4], sidx3, NS, 16)
        xs_pre, stats_s, gpart_s = _tc_smodel(
            xs_cur, a_tbl, mom12[1], mom34[0], mom34[1], bs1, u_cur, p['s2'])
        xs_bn = _tc_bn_apply(
            xs_pre, _bn_affine(stats_s, bn['xs']['g'], bn['xs']['b'],
                               float(NS)))
        gxs2 = _sc_gather1(xs_pre, src)
        th1, th2 = _tc_t1(gxs2, eap, p['t1'])
        tsum = _sc_scatter_sum([th1, th2], tidx3, NT, 10)
        xt_pre, stats_t, gpart_t = _tc_tmodel(xt_cur, tsum[0], tsum[1], bt1,
                                              u_cur, p['t2'])
        xt_bn = _tc_bn_apply(
            xt_pre, _bn_affine(stats_t, bn['xt']['g'], bn['xt']['b'],
                               float(NT)))
        u_new = _tc_gmodel(u_cur, gpart_s, gpart_t, p['g'])
        aff = _bn_affine(jnp.sum(bpart, axis=0), bn['e']['g'], bn['e']['b'],
                         float(E))
        ea = eap
        xs_cur, xt_cur, u_cur = xs_bn, xt_bn, u_new

    gxs, gxt = _sc_gather2(xs_cur, xt_cur, src, tgt)
    t = _tc_final(gxs, gxt, ea, be1, u_cur, params['last'], aff, noise)
    return (t.reshape(E), edge_index)


# speedup vs baseline: 3.2571x; 1.2921x over previous
import functools

import jax
import jax.numpy as jnp
from jax import lax
from jax.experimental import pallas as pl
from jax.experimental.pallas import tpu as pltpu
from jax.experimental.pallas import tpu_sc as plsc

NW = 32
RJ = 80
KJ = 5
CH = KJ * RJ
BE = 6400
BN = 4000


def _leaky(x):
    return jnp.where(x >= 0, x, 0.1 * x)


def _onehot16(idx_col):
    return (idx_col == lax.broadcasted_iota(jnp.int32, (1, 16), 1)).astype(jnp.float32)


def _sc_gather2(xs_tbl, xt_tbl, src, tgt):
    E = src.shape[0]
    DS = xs_tbl.shape[1]
    DT = xt_tbl.shape[1]
    perw = E // NW
    C2 = 1000
    npair = perw // C2 // 2
    mesh = plsc.VectorSubcoreMesh(core_axis_name="c", subcore_axis_name="s")

    @functools.partial(
        pl.kernel,
        out_type=(jax.ShapeDtypeStruct((E, DS), jnp.float32),
                  jax.ShapeDtypeStruct((E, DT), jnp.float32)),
        mesh=mesh,
        compiler_params=pltpu.CompilerParams(use_tc_tiling_on_sc=False),
        scratch_types=[pltpu.VMEM((C2,), jnp.int32),
                       pltpu.VMEM((C2,), jnp.int32),
                       pltpu.VMEM((C2,), jnp.int32),
                       pltpu.VMEM((C2,), jnp.int32),
                       pltpu.VMEM((C2, DS), jnp.float32),
                       pltpu.VMEM((C2, DS), jnp.float32),
                       pltpu.VMEM((C2, DT), jnp.float32),
                       pltpu.VMEM((C2, DT), jnp.float32),
                       pltpu.SemaphoreType.DMA,
                       pltpu.SemaphoreType.DMA,
                       pltpu.SemaphoreType.DMA,
                       pltpu.SemaphoreType.DMA],
    )
    def k(xs_hbm, xt_hbm, src_hbm, tgt_hbm, oxs_hbm, oxt_hbm,
          s0, s1, t0, t1, rs0, rs1, rt0, rt1, es0, es1, et0, et1):
        wid = lax.axis_index("s") * 2 + lax.axis_index("c")
        base = wid * perw

        def fire(i, ib, tb, rsb, rtb, ess, ets):
            off = base + i * C2
            pltpu.sync_copy(src_hbm.at[pl.ds(off, C2)], ib)
            pltpu.sync_copy(tgt_hbm.at[pl.ds(off, C2)], tb)
            pltpu.async_copy(xs_hbm.at[ib], rsb, ess)
            pltpu.async_copy(xt_hbm.at[tb], rtb, ets)

        def finish(i, ib, tb, rsb, rtb, ess, ets):
            off = base + i * C2
            pltpu.make_async_copy(xs_hbm.at[ib], rsb, ess).wait()
            pltpu.make_async_copy(xt_hbm.at[tb], rtb, ets).wait()
            pltpu.sync_copy(rsb, oxs_hbm.at[pl.ds(off, C2)])
            pltpu.sync_copy(rtb, oxt_hbm.at[pl.ds(off, C2)])

        fire(0, s0, t0, rs0, rt0, es0, et0)

        def body(ii, _):
            i = 2 * ii
            fire(i + 1, s1, t1, rs1, rt1, es1, et1)
            finish(i, s0, t0, rs0, rt0, es0, et0)
            fire(i + 2, s0, t0, rs0, rt0, es0, et0)
            finish(i + 1, s1, t1, rs1, rt1, es1, et1)
            return 0

        lax.fori_loop(0, npair - 1, body, 0)
        i = 2 * (npair - 1)
        fire(i + 1, s1, t1, rs1, rt1, es1, et1)
        finish(i, s0, t0, rs0, rt0, es0, et0)
        finish(i + 1, s1, t1, rs1, rt1, es1, et1)

    return k(xs_tbl, xt_tbl, src, tgt)


def _sc_gather1(tbl, idx):
    E = idx.shape[0]
    D = tbl.shape[1]
    perw = E // NW
    C1 = 1000
    npair = perw // C1 // 2
    mesh = plsc.VectorSubcoreMesh(core_axis_name="c", subcore_axis_name="s")

    @functools.partial(
        pl.kernel,
        out_type=jax.ShapeDtypeStruct((E, D), jnp.float32),
        mesh=mesh,
        compiler_params=pltpu.CompilerParams(use_tc_tiling_on_sc=False),
        scratch_types=[pltpu.VMEM((C1,), jnp.int32),
                       pltpu.VMEM((C1,), jnp.int32),
                       pltpu.VMEM((C1, D), jnp.float32),
                       pltpu.VMEM((C1, D), jnp.float32),
                       pltpu.SemaphoreType.DMA,
                       pltpu.SemaphoreType.DMA],
    )
    def k(tbl_hbm, idx_hbm, out_hbm, i0, i1, r0, r1, e0, e1):
        wid = lax.axis_index("s") * 2 + lax.axis_index("c")
        base = wid * perw

        def fire(i, ib, rb, sem):
            pltpu.sync_copy(idx_hbm.at[pl.ds(base + i * C1, C1)], ib)
            pltpu.async_copy(tbl_hbm.at[ib], rb, sem)

        def finish(i, ib, rb, sem):
            pltpu.make_async_copy(tbl_hbm.at[ib], rb, sem).wait()
            pltpu.sync_copy(rb, out_hbm.at[pl.ds(base + i * C1, C1)])

        fire(0, i0, r0, e0)

        def body(ii, _):
            i = 2 * ii
            fire(i + 1, i1, r1, e1)
            finish(i, i0, r0, e0)
            fire(i + 2, i0, r0, e0)
            finish(i + 1, i1, r1, e1)
            return 0

        lax.fori_loop(0, npair - 1, body, 0)
        i = 2 * (npair - 1)
        fire(i + 1, i1, r1, e1)
        finish(i, i0, r0, e0)
        finish(i + 1, i1, r1, e1)

    return k(tbl, idx)


SKJ = 8
SCH = SKJ * RJ
SNIT = 78
SPAD = 632


def _sc_scatter_sum(rows_list, idx3, nseg, width):
    M = len(rows_list)
    E = rows_list[0].shape[0]
    perw = E // NW
    zslc = 6256 if nseg == 100000 else nseg // 16
    mesh = plsc.VectorSubcoreMesh(core_axis_name="c", subcore_axis_name="s")
    zer = jnp.zeros((nseg, width), jnp.float32)

    @functools.partial(
        pl.kernel,
        out_type=jax.ShapeDtypeStruct((M, 2, nseg, width), jnp.float32),
        mesh=mesh,
        compiler_params=pltpu.CompilerParams(use_tc_tiling_on_sc=False),
        scratch_types=[pltpu.VMEM((SKJ, RJ), jnp.int32),
                       pltpu.VMEM((SKJ, RJ), jnp.int32),
                       pltpu.VMEM((SCH, width), jnp.float32),
                       pltpu.VMEM((SCH, width), jnp.float32),
                       pltpu.VMEM_SHARED((nseg, width), jnp.float32),
                       pltpu.SemaphoreType.DMA,
                       pltpu.SemaphoreType.DMA,
                       pltpu.SemaphoreType.DMA,
                       pltpu.SemaphoreType.DMA],
    )
    def k(*refs):
        rows_hbms = refs[:M]
        zer_hbm = refs[M]
        idx_hbm = refs[M + 1]
        out_hbm = refs[M + 2]
        i0, i1, r0, r1, acc, lr0, lr1, sa0, sa1 = refs[M + 3:]
        cid = lax.axis_index("c")
        sid = lax.axis_index("s")
        wid = sid * 2 + cid
        base = wid * perw
        zoff = jnp.minimum(sid * zslc, nseg - zslc)
        for m in range(M):
            rows = rows_hbms[m]

            def load(i, ib, rb, sem):
                pltpu.sync_copy(idx_hbm.at[wid, pl.ds(i * SKJ, SKJ)], ib)
                pltpu.async_copy(rows.at[pl.ds(base + i * SCH, SCH)], rb, sem)

            def scat(ib, rb, lsem, ssem):
                pltpu.make_async_copy(
                    rows.at[pl.ds(0, SCH)], rb, lsem).wait()
                for j in range(SKJ):
                    pltpu.async_copy(rb.at[pl.ds(j * RJ, RJ)],
                                     acc.at[ib.at[j]], ssem, add=True)

            def drain(ib, rb, ssem):
                for j in range(SKJ):
                    pltpu.make_async_copy(rb.at[pl.ds(j * RJ, RJ)],
                                          acc.at[ib.at[j]], ssem).wait()

            pltpu.sync_copy(zer_hbm.at[pl.ds(zoff, zslc)],
                            acc.at[pl.ds(zoff, zslc)])
            plsc.subcore_barrier()
            load(0, i0, r0, lr0)

            def body(ii, _):
                i = 2 * ii
                load(i + 1, i1, r1, lr1)
                scat(i0, r0, lr0, sa0)
                drain(i0, r0, sa0)
                load(i + 2, i0, r0, lr0)
                scat(i1, r1, lr1, sa1)
                drain(i1, r1, sa1)
                return 0

            lax.fori_loop(0, SNIT // 2 - 1, body, 0)
            i = SNIT - 2
            load(i + 1, i1, r1, lr1)
            scat(i0, r0, lr0, sa0)
            drain(i0, r0, sa0)
            scat(i1, r1, lr1, sa1)
            drain(i1, r1, sa1)
            pltpu.sync_copy(idx_hbm.at[wid, pl.ds(SNIT * SKJ, SKJ)], i0)
            pltpu.sync_copy(rows.at[pl.ds(base + SNIT * SCH, RJ)],
                            r0.at[pl.ds(0, RJ)])
            pltpu.sync_copy(r0.at[pl.ds(0, RJ)], acc.at[i0.at[0]],
                            add=True)
            plsc.subcore_barrier()
            pltpu.sync_copy(acc.at[pl.ds(zoff, zslc)],
                            out_hbm.at[m, cid, pl.ds(zoff, zslc)])
            plsc.subcore_barrier()

    return k(*rows_list, zer, idx3)


def _tc_edge(gxs, gxt, ea, be1, u, pe, ps1, aff):
    E = gxs.shape[0]
    G = E // BE

    def body(gxs_r, gxt_r, ea_r, be_r, u_r, w1_r, c1_r, w2_r, c2_r,
             v1_r, d1_r, v2_r, d2_r, aff_r,
             eap_r, p1_r, p2_r, bp_r):
        ea_b = ea_r[...] * aff_r[0:1, :] + aff_r[1:2, :]
        ue = _onehot16(be_r[...]) @ u_r[...]
        x = gxs_r[:, 0:10]
        xt = gxt_r[:, 0:5]
        h = (x @ w1_r[0:10, :] + xt @ w1_r[10:15, :] + ea_b @ w1_r[15:25, :]
             + ue @ w1_r[25:35, :] + c1_r[...])
        eap = _leaky(h) @ w2_r[...] + c2_r[...]
        eap_r[...] = eap
        bp_r[0, 0, :] = jnp.sum(eap, 0)
        bp_r[0, 1, :] = jnp.sum(eap * eap, 0)
        sh = _leaky(xt @ v1_r[0:5, :] + eap @ v1_r[5:15, :] + d1_r[...])
        s_out = sh @ v2_r[...] + d2_r[...]
        p1 = jnp.concatenate([s_out, jnp.ones((BE, 1), jnp.float32)], axis=1)
        p1_r[...] = p1
        p2_r[...] = p1 * p1

    full = lambda s: pl.BlockSpec(s, lambda i: (0,) * len(s))
    row = lambda w: pl.BlockSpec((BE, w), lambda i: (i, 0))
    out = pl.pallas_call(
        body,
        grid=(G,),
        in_specs=[row(16), row(8), row(10), row(1), full((16, 10)),
                  full((35, 10)), full((1, 10)), full((10, 10)), full((1, 10)),
                  full((15, 15)), full((1, 15)), full((15, 15)), full((1, 15)),
                  full((2, 10))],
        out_specs=[row(10), row(16), row(16),
                   pl.BlockSpec((1, 2, 10), lambda i: (i, 0, 0))],
        out_shape=[jax.ShapeDtypeStruct((E, 10), jnp.float32)]
                  + [jax.ShapeDtypeStruct((E, 16), jnp.float32)] * 2
                  + [jax.ShapeDtypeStruct((G, 2, 10), jnp.float32)],
    )(gxs, gxt, ea, be1, u,
      pe['l1']['W'], pe['l1']['b'].reshape(1, 10), pe['l2']['W'],
      pe['l2']['b'].reshape(1, 10),
      ps1['l1']['W'], ps1['l1']['b'].reshape(1, 15), ps1['l2']['W'],
      ps1['l2']['b'].reshape(1, 15), aff)
    return out


def _tc_amean(mom1):
    NS = mom1.shape[1]
    G = NS // BN

    def body(m_r, a_r):
        m = m_r[0] + m_r[1]
        n = m[:, 15:16]
        a_r[...] = jnp.concatenate(
            [m[:, 0:15] / jnp.maximum(n, 1.0), n], axis=1)

    return pl.pallas_call(
        body,
        grid=(G,),
        in_specs=[pl.BlockSpec((2, BN, 16), lambda i: (0, i, 0))],
        out_specs=pl.BlockSpec((BN, 16), lambda i: (i, 0)),
        out_shape=jax.ShapeDtypeStruct((NS, 16), jnp.float32),
    )(mom1)


def _tc_center(p1, arows):
    E = p1.shape[0]
    G = E // BE

    def body(p_r, a_r, q3_r, q4_r):
        cen = p_r[...] - a_r[...]
        c2 = cen * cen
        q3_r[...] = c2 * cen
        q4_r[...] = c2 * c2

    row = lambda w: pl.BlockSpec((BE, w), lambda i: (i, 0))
    return pl.pallas_call(
        body,
        grid=(G,),
        in_specs=[row(16), row(16)],
        out_specs=[row(16), row(16)],
        out_shape=[jax.ShapeDtypeStruct((E, 16), jnp.float32)] * 2,
    )(p1, arows)


def _tc_smodel(xs, a_tbl, m2p, q3p, q4p, bs1, u, ps2):
    NS = xs.shape[0]
    G = NS // BN

    def body(xs_r, a_r, m2_r, q3_r, q4_r, bs_r, u_r, w1_r, c1_r, w2_r, c2_r,
             pre_r, st_r, gp_r):
        i = pl.program_id(0)

        @pl.when(i == 0)
        def _():
            st_r[...] = jnp.zeros((2, 16), jnp.float32)
            gp_r[...] = jnp.zeros((16, 16), jnp.float32)

        n = a_r[:, 15:16]
        nc = jnp.maximum(n, 1.0)
        a = a_r[:, 0:15]
        x = xs_r[:, 0:10]
        m2 = (m2_r[0] + m2_r[1])[:, 0:15] / nc
        m3 = (q3_r[0] + q3_r[1])[:, 0:15] / nc
        m4 = (q4_r[0] + q4_r[1])[:, 0:15] / nc
        b = jnp.sqrt(1e-6 + jnp.maximum(m2 - a * a, 0.0))
        c = m3 / (b * b * b)
        d = m4 / ((b * b) * (b * b))
        oh = _onehot16(bs_r[...])
        ue = oh @ u_r[...]
        h = (x @ w1_r[0:10, :] + n @ w1_r[10:11, :] + a @ w1_r[11:26, :]
             + b @ w1_r[26:41, :] + c @ w1_r[41:56, :] + d @ w1_r[56:71, :]
             + ue @ w1_r[71:81, :] + c1_r[...])
        xp = _leaky(h) @ w2_r[...] + c2_r[...]
        pre_r[...] = jnp.concatenate(
            [xp, jnp.zeros((BN, 6), jnp.float32)], axis=1)
        st_r[0:1, 0:10] = st_r[0:1, 0:10] + jnp.sum(xp, 0, keepdims=True)
        st_r[1:2, 0:10] = st_r[1:2, 0:10] + jnp.sum(xp * xp, 0, keepdims=True)
        gp_r[:, 0:10] = gp_r[:, 0:10] + lax.dot_general(
            oh, xp, (((0,), (0,)), ((), ())),
            preferred_element_type=jnp.float32)
        gp_r[:, 10:11] = gp_r[:, 10:11] + jnp.sum(oh, 0)[:, None]

    full = lambda s: pl.BlockSpec(s, lambda i: (0,) * len(s))
    row = lambda w: pl.BlockSpec((BN, w), lambda i: (i, 0))
    mom_s = pl.BlockSpec((2, BN, 16), lambda i: (0, i, 0))
    return pl.pallas_call(
        body,
        grid=(G,),
        in_specs=[row(16), row(16), mom_s, mom_s, mom_s, row(1),
                  full((16, 10)),
                  full((81, 10)), full((1, 10)), full((10, 10)), full((1, 10))],
        out_specs=[row(16), full((2, 16)), full((16, 16))],
        out_shape=[jax.ShapeDtypeStruct((NS, 16), jnp.float32),
                   jax.ShapeDtypeStruct((2, 16), jnp.float32),
                   jax.ShapeDtypeStruct((16, 16), jnp.float32)],
    )(xs, a_tbl, m2p, q3p, q4p, bs1, u,
      ps2['l1']['W'], ps2['l1']['b'].reshape(1, 10), ps2['l2']['W'],
      ps2['l2']['b'].reshape(1, 10))


def _tc_bn_apply(xp, aff, ow):
    N, IW = xp.shape
    W = aff.shape[1]
    G = N // BN

    def body(x_r, a_r, o_r):
        v = x_r[:, 0:W] * a_r[0:1, :] + a_r[1:2, :]
        o_r[...] = jnp.concatenate(
            [v, jnp.zeros((BN, ow - W), jnp.float32)], axis=1)

    return pl.pallas_call(
        body,
        grid=(G,),
        in_specs=[pl.BlockSpec((BN, IW), lambda i: (i, 0)),
                  pl.BlockSpec((2, W), lambda i: (0, 0))],
        out_specs=pl.BlockSpec((BN, ow), lambda i: (i, 0)),
        out_shape=jax.ShapeDtypeStruct((N, ow), jnp.float32),
    )(xp, aff)


def _tc_t1(gxs2, eap, pt1, wc):
    E = gxs2.shape[0]
    G = E // BE

    def body(x_r, e_r, w1_r, c1_r, wc_r, o_r):
        h = _leaky(x_r[:, 0:10] @ w1_r[0:10, :] + e_r[...] @ w1_r[10:20, :]
                   + c1_r[...])
        w = h @ wc_r[...]
        o_r[...] = jnp.concatenate(
            [w, jnp.ones((BE, 1), jnp.float32),
             jnp.zeros((BE, 2), jnp.float32)], axis=1)

    full = lambda s: pl.BlockSpec(s, lambda i: (0,) * len(s))
    row = lambda w: pl.BlockSpec((BE, w), lambda i: (i, 0))
    return pl.pallas_call(
        body,
        grid=(G,),
        in_specs=[row(16), row(10), full((20, 20)), full((1, 20)),
                  full((20, 5))],
        out_specs=row(8),
        out_shape=jax.ShapeDtypeStruct((E, 8), jnp.float32),
    )(gxs2, eap, pt1['l1']['W'], pt1['l1']['b'].reshape(1, 20), wc)


def _tc_tmodel(xt, tsum, bt1, u, pt2, bc):
    NT = xt.shape[0]
    G = NT // BN

    def body(xt_r, ts_r, bt_r, u_r, w1_r, c1_r, w2_r, c2_r, bc_r,
             pre_r, st_r, gp_r):
        i = pl.program_id(0)

        @pl.when(i == 0)
        def _():
            st_r[...] = jnp.zeros((2, 8), jnp.float32)
            gp_r[...] = jnp.zeros((16, 8), jnp.float32)

        ts = ts_r[0] + ts_r[1]
        av = ts[:, 0:5] + ts[:, 5:6] * bc_r[...]
        oh = _onehot16(bt_r[...])
        ue = oh @ u_r[...]
        x = xt_r[:, 0:5]
        h = (x @ w1_r[0:5, :] + av + ue @ w1_r[25:35, :]
             + c1_r[...])
        xp = _leaky(h) @ w2_r[...] + c2_r[...]
        pre_r[...] = jnp.concatenate(
            [xp, jnp.zeros((BN, 3), jnp.float32)], axis=1)
        st_r[0:1, 0:5] = st_r[0:1, 0:5] + jnp.sum(xp, 0, keepdims=True)
        st_r[1:2, 0:5] = st_r[1:2, 0:5] + jnp.sum(xp * xp, 0, keepdims=True)
        gp_r[:, 0:5] = gp_r[:, 0:5] + lax.dot_general(
            oh, xp, (((0,), (0,)), ((), ())),
            preferred_element_type=jnp.float32)
        gp_r[:, 5:6] = gp_r[:, 5:6] + jnp.sum(oh, 0)[:, None]

    full = lambda s: pl.BlockSpec(s, lambda i: (0,) * len(s))
    row = lambda w: pl.BlockSpec((BN, w), lambda i: (i, 0))
    return pl.pallas_call(
        body,
        grid=(G,),
        in_specs=[row(8), pl.BlockSpec((2, BN, 8), lambda i: (0, i, 0)),
                  row(1), full((16, 10)),
                  full((35, 5)), full((1, 5)), full((5, 5)), full((1, 5)),
                  full((1, 5))],
        out_specs=[row(8), full((2, 8)), full((16, 8))],
        out_shape=[jax.ShapeDtypeStruct((NT, 8), jnp.float32),
                   jax.ShapeDtypeStruct((2, 8), jnp.float32),
                   jax.ShapeDtypeStruct((16, 8), jnp.float32)],
    )(xt, tsum, bt1, u,
      pt2['l1']['W'], pt2['l1']['b'].reshape(1, 5), pt2['l2']['W'],
      pt2['l2']['b'].reshape(1, 5), bc)


def _tc_gmodel(u, gpart_s, gpart_t, pg):
    def body(u_r, gs_r, gt_r, g1_r, e1_r, g2_r, e2_r, uo_r):
        ms = gs_r[:, 0:10] / jnp.maximum(gs_r[:, 10:11], 1.0)
        mt = gt_r[:, 0:5] / jnp.maximum(gt_r[:, 5:6], 1.0)
        hg = (u_r[...] @ g1_r[0:10, :] + ms @ g1_r[10:20, :]
              + mt @ g1_r[20:25, :] + e1_r[...])
        uo_r[...] = _leaky(hg) @ g2_r[...] + e2_r[...]

    return pl.pallas_call(
        body,
        out_shape=jax.ShapeDtypeStruct((16, 10), jnp.float32),
    )(u, gpart_s, gpart_t,
      pg['l1']['W'], pg['l1']['b'].reshape(1, 10), pg['l2']['W'],
      pg['l2']['b'].reshape(1, 10))


def _tc_final(gxs, gxt, ea, be1, u, pl_last, aff, noise):
    E = gxs.shape[0]
    G = E // BE

    def body(gxs_r, gxt_r, ea_r, be_r, u_r, w1_r, c1_r, w2_r, c2_r,
             aff_r, nz_r, t_r):
        ea_b = ea_r[...] * aff_r[0:1, :] + aff_r[1:2, :]
        ue = _onehot16(be_r[...]) @ u_r[...]
        h = (gxs_r[:, 0:10] @ w1_r[0:10, :] + gxt_r[:, 0:5] @ w1_r[10:15, :]
             + ea_b @ w1_r[15:25, :] + ue @ w1_r[25:35, :] + c1_r[...])
        o = _leaky(h) @ w2_r[...] + c2_r[...]
        mx = jnp.max(o, axis=1, keepdims=True)
        e = jnp.exp(o - mx)
        cls = lax.broadcasted_iota(jnp.int32, (1, 5), 1).astype(jnp.float32)
        t = jnp.sum(e * cls, axis=1, keepdims=True) \
            / jnp.sum(e, axis=1, keepdims=True)
        t = t + nz_r[...]
        it = jnp.floor(t)
        z = 20.0 * (t - 0.5 - it)
        t_r[...] = it + 1.0 / (1.0 + jnp.exp(-z))

    full = lambda s: pl.BlockSpec(s, lambda i: (0,) * len(s))
    row = lambda w: pl.BlockSpec((BE, w), lambda i: (i, 0))
    return pl.pallas_call(
        body,
        grid=(G,),
        in_specs=[row(16), row(8), row(10), row(1), full((16, 10)),
                  full((35, 5)), full((1, 5)), full((5, 5)), full((1, 5)),
                  full((2, 10)), row(1)],
        out_specs=row(1),
        out_shape=jax.ShapeDtypeStruct((E, 1), jnp.float32),
    )(gxs, gxt, ea, be1, u, pl_last['l1']['W'],
      pl_last['l1']['b'].reshape(1, 5), pl_last['l2']['W'],
      pl_last['l2']['b'].reshape(1, 5), aff, noise)


def _bn_affine(stats, g, b, n):
    w = g.shape[0]
    mean = stats[0, :w] / n
    var = stats[1, :w] / n - mean * mean
    sc = g * lax.rsqrt(var + 1e-5)
    return jnp.stack([sc, b - mean * sc])


def kernel(x_s, x_t, edge_index, edge_attr, u, batch_e, batch_s, batch_t,
           params):
    E = edge_attr.shape[0]
    NS = x_s.shape[0]
    NT = x_t.shape[0]
    src = edge_index[0]
    tgt = edge_index[1]
    nrow = E // NW // RJ
    sidx3 = jnp.pad(src.reshape(NW, nrow, RJ),
                    ((0, 0), (0, SPAD - nrow), (0, 0)))
    tidx3 = jnp.pad(tgt.reshape(NW, nrow, RJ),
                    ((0, 0), (0, SPAD - nrow), (0, 0)))
    be1 = batch_e.reshape(E, 1)
    bs1 = batch_s.reshape(NS, 1)
    bt1 = batch_t.reshape(NT, 1)
    noise = (0.3 * (jax.random.uniform(jax.random.key(1234), (E,),
                                       jnp.float32) - 0.5)).reshape(E, 1)

    ea = edge_attr
    aff = jnp.stack([jnp.ones((10,), jnp.float32),
                     jnp.zeros((10,), jnp.float32)])
    xs_cur = jnp.pad(x_s, ((0, 0), (0, 6)))
    xt_cur = jnp.pad(x_t, ((0, 0), (0, 3)))
    u_cur = u

    for i in range(4):
        p = params['blocks'][i]
        bn = params['bns'][i]
        gxs, gxt = _sc_gather2(xs_cur, xt_cur, src, tgt)
        eap, p1, p2, bpart = _tc_edge(
            gxs, gxt, ea, be1, u_cur, p['edge'], p['s1'], aff)
        mom12 = _sc_scatter_sum([p1, p2], sidx3, NS, 16)
        a_tbl = _tc_amean(mom12[0])
        arows = _sc_gather1(a_tbl, src)
        q3, q4 = _tc_center(p1, arows)
        mom34 = _sc_scatter_sum([q3, q4], sidx3, NS, 16)
        xs_pre, stats_s, gpart_s = _tc_smodel(
            xs_cur, a_tbl, mom12[1], mom34[0], mom34[1], bs1, u_cur, p['s2'])
        xs_bn = _tc_bn_apply(
            xs_pre, _bn_affine(stats_s, bn['xs']['g'], bn['xs']['b'],
                               float(NS)), 16)
        gxs2 = _sc_gather1(xs_pre, src)
        w1b = p['t2']['l1']['W'][5:25, :]
        wc = p['t1']['l2']['W'] @ w1b
        bc = (p['t1']['l2']['b'].reshape(1, 20) @ w1b)
        th = _tc_t1(gxs2, eap, p['t1'], wc)
        tsum = _sc_scatter_sum([th], tidx3, NT, 8)
        xt_pre, stats_t, gpart_t = _tc_tmodel(xt_cur, tsum[0], bt1,
                                              u_cur, p['t2'], bc)
        xt_bn = _tc_bn_apply(
            xt_pre, _bn_affine(stats_t, bn['xt']['g'], bn['xt']['b'],
                               float(NT)), 8)
        u_new = _tc_gmodel(u_cur, gpart_s, gpart_t, p['g'])
        aff = _bn_affine(jnp.sum(bpart, axis=0), bn['e']['g'], bn['e']['b'],
                         float(E))
        ea = eap
        xs_cur, xt_cur, u_cur = xs_bn, xt_bn, u_new

    gxs, gxt = _sc_gather2(xs_cur, xt_cur, src, tgt)
    t = _tc_final(gxs, gxt, ea, be1, u_cur, params['last'], aff, noise)
    return (t.reshape(E), edge_index)
